# Initial kernel scaffold; baseline (speedup 1.0000x reference)
import jax
import jax.numpy as jnp
from jax.experimental import pallas as pl


def kernel(input, adj, W, a):
    raise NotImplementedError("write your pallas kernel here")


# speedup vs baseline: 1808.2932x; 1808.2932x over previous
import functools

import jax
import jax.numpy as jnp
from jax.experimental import pallas as pl
from jax.experimental.pallas import tpu as pltpu


def _gat_block_kernel(x_ref, adj_ref, w_ref, a_ref, o_ref, h_ref):
    i = pl.program_id(1)
    fo = h_ref.shape[-1]
    blk = adj_ref.shape[0]

    @pl.when(i == 0)
    def _compute_h():
        h_ref[...] = jnp.dot(
            x_ref[0], w_ref[...], preferred_element_type=jnp.float32
        )

    h = h_ref[...]
    a1 = a_ref[:fo, :]
    a2 = a_ref[fo:, :]

    h_blk = h_ref[pl.ds(i * blk, blk), :]
    s1 = jax.lax.dot_general(
        h_blk, a1, (((1,), (0,)), ((), ())),
        preferred_element_type=jnp.float32,
    )
    s2t = jax.lax.dot_general(
        a2, h, (((0,), (1,)), ((), ())),
        preferred_element_type=jnp.float32,
    )
    scores = s1 + s2t
    neg = jnp.where(scores >= 0.0, -scores, -0.2 * scores)
    ee = jnp.where(adj_ref[...] != 0, jnp.exp(neg), 0.0)

    rowsum = jnp.sum(ee, axis=1, keepdims=True)
    agg = jnp.dot(ee, h, preferred_element_type=jnp.float32)
    hp = agg / rowsum
    o_ref[0] = jnp.where(hp > 0.0, hp, jnp.exp(hp) - 1.0)


@functools.partial(jax.jit, static_argnames=())
def kernel(input, adj, W, a):
    b, n, f = input.shape
    fo = W.shape[1]
    blk = 256
    nblk = n // blk

    grid = (b, nblk)
    out = pl.pallas_call(
        _gat_block_kernel,
        grid=grid,
        in_specs=[
            pl.BlockSpec((1, n, f), lambda ib, i: (ib, 0, 0)),
            pl.BlockSpec((blk, n), lambda ib, i: (i, 0)),
            pl.BlockSpec((f, fo), lambda ib, i: (0, 0)),
            pl.BlockSpec((2 * fo, 1), lambda ib, i: (0, 0)),
        ],
        out_specs=pl.BlockSpec((1, blk, fo), lambda ib, i: (ib, i, 0)),
        out_shape=jax.ShapeDtypeStruct((b, n, fo), jnp.float32),
        scratch_shapes=[pltpu.VMEM((n, fo), jnp.float32)],
    )(input, adj, W, a)
    return out


# speedup vs baseline: 2236.2158x; 1.2366x over previous
import functools

import jax
import jax.numpy as jnp
from jax.experimental import pallas as pl
from jax.experimental.pallas import tpu as pltpu


def _gat_block_kernel(
    x_ref, adj_ref, w_ref, a_ref, o_ref,
    h_ref, s1_ref, s2_ref, u1_ref, u2_ref, v1_ref, v2_ref,
):
    i = pl.program_id(0)
    nb = x_ref.shape[0]
    fo = h_ref.shape[-1]
    blk = adj_ref.shape[0]

    @pl.when(i == 0)
    def _precompute():
        for b in range(nb):
            h = jnp.dot(x_ref[b], w_ref[...], preferred_element_type=jnp.float32)
            h_ref[b] = h
            s1 = jax.lax.dot_general(
                h, a_ref[:fo, :], (((1,), (0,)), ((), ())),
                preferred_element_type=jnp.float32,
            )
            s2t = jax.lax.dot_general(
                a_ref[fo:, :], h, (((0,), (1,)), ((), ())),
                preferred_element_type=jnp.float32,
            )
            s1_ref[b] = s1
            s2_ref[b] = s2t
            u1_ref[b] = jnp.exp(-s1)
            u2_ref[b] = jnp.exp(-0.2 * s1)
            v1_ref[b] = jnp.exp(-s2t)
            v2_ref[b] = jnp.exp(-0.2 * s2t)

    mask = adj_ref[...] != 0
    for b in range(nb):
        s1b = s1_ref[b, pl.ds(i * blk, blk), :]
        nonneg = s1b + s2_ref[b] >= 0.0
        prod = jnp.where(
            nonneg,
            u1_ref[b, pl.ds(i * blk, blk), :] * v1_ref[b],
            u2_ref[b, pl.ds(i * blk, blk), :] * v2_ref[b],
        )
        ee = jnp.where(mask, prod, 0.0)
        rowsum = jnp.sum(ee, axis=1, keepdims=True)
        agg = jnp.dot(ee, h_ref[b], preferred_element_type=jnp.float32)
        hp = agg / rowsum
        o_ref[b] = jnp.where(hp > 0.0, hp, jnp.exp(hp) - 1.0)


@functools.partial(jax.jit, static_argnames=())
def kernel(input, adj, W, a):
    b, n, f = input.shape
    fo = W.shape[1]
    blk = 256
    nblk = n // blk

    out = pl.pallas_call(
        _gat_block_kernel,
        grid=(nblk,),
        in_specs=[
            pl.BlockSpec((b, n, f), lambda i: (0, 0, 0)),
            pl.BlockSpec((blk, n), lambda i: (i, 0)),
            pl.BlockSpec((f, fo), lambda i: (0, 0)),
            pl.BlockSpec((2 * fo, 1), lambda i: (0, 0)),
        ],
        out_specs=pl.BlockSpec((b, blk, fo), lambda i: (0, i, 0)),
        out_shape=jax.ShapeDtypeStruct((b, n, fo), jnp.float32),
        scratch_shapes=[
            pltpu.VMEM((b, n, fo), jnp.float32),
            pltpu.VMEM((b, n, 1), jnp.float32),
            pltpu.VMEM((b, 1, n), jnp.float32),
            pltpu.VMEM((b, n, 1), jnp.float32),
            pltpu.VMEM((b, n, 1), jnp.float32),
            pltpu.VMEM((b, 1, n), jnp.float32),
            pltpu.VMEM((b, 1, n), jnp.float32),
        ],
    )(input, adj, W, a)
    return out


# speedup vs baseline: 2333.3574x; 1.0434x over previous
import functools

import jax
import jax.numpy as jnp
from jax.experimental import pallas as pl
from jax.experimental.pallas import tpu as pltpu


def _gat_block_kernel(
    x_ref, adj_ref, w_ref, a_ref, o_ref,
    h_ref, u1_ref, u2_ref, v1_ref, v2_ref,
):
    i = pl.program_id(0)
    nb = x_ref.shape[0]
    fo = h_ref.shape[-1]
    blk = adj_ref.shape[0]

    @pl.when(i == 0)
    def _precompute():
        for b in range(nb):
            h = jnp.dot(x_ref[b], w_ref[...], preferred_element_type=jnp.float32)
            h_ref[b] = h
            s1 = jax.lax.dot_general(
                h, a_ref[:fo, :], (((1,), (0,)), ((), ())),
                preferred_element_type=jnp.float32,
            )
            s2t = jax.lax.dot_general(
                a_ref[fo:, :], h, (((0,), (1,)), ((), ())),
                preferred_element_type=jnp.float32,
            )
            u1_ref[b] = jnp.exp(-s1)
            u2_ref[b] = jnp.exp(-0.2 * s1)
            v1_ref[b] = jnp.exp(-s2t)
            v2_ref[b] = jnp.exp(-0.2 * s2t)

    mask = adj_ref[...] != 0
    for b in range(nb):
        prod = jnp.minimum(
            u1_ref[b, pl.ds(i * blk, blk), :] * v1_ref[b],
            u2_ref[b, pl.ds(i * blk, blk), :] * v2_ref[b],
        )
        ee = jnp.where(mask, prod, 0.0)
        rowsum = jnp.sum(ee, axis=1, keepdims=True)
        agg = jnp.dot(ee, h_ref[b], preferred_element_type=jnp.float32)
        hp = agg / rowsum
        o_ref[b] = jnp.where(hp > 0.0, hp, jnp.exp(hp) - 1.0)


@functools.partial(jax.jit, static_argnames=())
def kernel(input, adj, W, a):
    b, n, f = input.shape
    fo = W.shape[1]
    blk = 256
    nblk = n // blk

    out = pl.pallas_call(
        _gat_block_kernel,
        grid=(nblk,),
        in_specs=[
            pl.BlockSpec((b, n, f), lambda i: (0, 0, 0)),
            pl.BlockSpec((blk, n), lambda i: (i, 0)),
            pl.BlockSpec((f, fo), lambda i: (0, 0)),
            pl.BlockSpec((2 * fo, 1), lambda i: (0, 0)),
        ],
        out_specs=pl.BlockSpec((b, blk, fo), lambda i: (0, i, 0)),
        out_shape=jax.ShapeDtypeStruct((b, n, fo), jnp.float32),
        scratch_shapes=[
            pltpu.VMEM((b, n, fo), jnp.float32),
            pltpu.VMEM((b, n, 1), jnp.float32),
            pltpu.VMEM((b, n, 1), jnp.float32),
            pltpu.VMEM((b, 1, n), jnp.float32),
            pltpu.VMEM((b, 1, n), jnp.float32),
        ],
    )(input, adj, W, a)
    return out


# speedup vs baseline: 2478.8171x; 1.0623x over previous
import functools

import jax
import jax.numpy as jnp
from jax.experimental import pallas as pl
from jax.experimental.pallas import tpu as pltpu


def _gat_block_kernel(
    x_ref, adj_ref, w_ref, a_ref, o_ref,
    h_ref, u1_ref, u2_ref, v1_ref, v2_ref,
):
    i = pl.program_id(0)
    nb = x_ref.shape[0]
    fo = h_ref.shape[-1]
    blk = adj_ref.shape[0]

    @pl.when(i == 0)
    def _precompute():
        for b in range(nb):
            h = jnp.dot(x_ref[b], w_ref[...], preferred_element_type=jnp.float32)
            h_ref[b] = h
            s1 = jax.lax.dot_general(
                h, a_ref[:fo, :], (((1,), (0,)), ((), ())),
                preferred_element_type=jnp.float32,
            )
            s2t = jax.lax.dot_general(
                a_ref[fo:, :], h, (((0,), (1,)), ((), ())),
                preferred_element_type=jnp.float32,
            )
            u1_ref[b] = jnp.exp(-s1)
            u2_ref[b] = jnp.exp(-0.2 * s1)
            v1_ref[b] = jnp.exp(-s2t)
            v2_ref[b] = jnp.exp(-0.2 * s2t)

    mask = adj_ref[...] != 0
    for b in range(nb):
        prod = jnp.minimum(
            u1_ref[b, pl.ds(i * blk, blk), :] * v1_ref[b],
            u2_ref[b, pl.ds(i * blk, blk), :] * v2_ref[b],
        )
        ee = jnp.where(mask, prod, 0.0)
        rowsum = jnp.sum(ee, axis=1, keepdims=True)
        agg = jnp.dot(ee, h_ref[b], preferred_element_type=jnp.float32)
        hp = agg / rowsum
        o_ref[b] = jnp.where(hp > 0.0, hp, jnp.exp(hp) - 1.0)


@functools.partial(jax.jit, static_argnames=())
def kernel(input, adj, W, a):
    b, n, f = input.shape
    fo = W.shape[1]
    blk = 512
    nblk = n // blk

    out = pl.pallas_call(
        _gat_block_kernel,
        grid=(nblk,),
        in_specs=[
            pl.BlockSpec((b, n, f), lambda i: (0, 0, 0)),
            pl.BlockSpec((blk, n), lambda i: (i, 0)),
            pl.BlockSpec((f, fo), lambda i: (0, 0)),
            pl.BlockSpec((2 * fo, 1), lambda i: (0, 0)),
        ],
        out_specs=pl.BlockSpec((b, blk, fo), lambda i: (0, i, 0)),
        out_shape=jax.ShapeDtypeStruct((b, n, fo), jnp.float32),
        scratch_shapes=[
            pltpu.VMEM((b, n, fo), jnp.float32),
            pltpu.VMEM((b, n, 1), jnp.float32),
            pltpu.VMEM((b, n, 1), jnp.float32),
            pltpu.VMEM((b, 1, n), jnp.float32),
            pltpu.VMEM((b, 1, n), jnp.float32),
        ],
    )(input, adj, W, a)
    return out
